# Initial kernel scaffold; baseline (speedup 1.0000x reference)
import jax
import jax.numpy as jnp
from jax.experimental import pallas as pl


def kernel(inputs, targets, table):
    raise NotImplementedError("write your pallas kernel here")


# speedup vs baseline: 1.4122x; 1.4122x over previous
import functools

import jax
import jax.numpy as jnp
from jax import lax
from jax.experimental import pallas as pl
from jax.experimental.pallas import tpu as pltpu
from jax.experimental.pallas import tpu_sc as plsc

VOCAB = 1000
B = 1024
T = 50
N = B * T
NC = 2
NS = 16
L = 16
NW = NC * NS
TOK_PER_W = N // NW
CHUNK = 80
NCHUNK = TOK_PER_W // CHUNK
LSE_PAD = 1024


def _lse_body(table_ref, out_ref):
    x = table_ref[...]
    m = jnp.max(x, axis=1, keepdims=True)
    s = jnp.sum(jnp.exp(x - m), axis=1, keepdims=True)
    out_ref[...] = jnp.log(s) + m


_lse_call = pl.pallas_call(
    _lse_body,
    out_shape=jax.ShapeDtypeStruct((VOCAB, 1), jnp.float32),
)


@functools.partial(
    pl.kernel,
    out_type=(
        jax.ShapeDtypeStruct((N, VOCAB), jnp.float32),
        jax.ShapeDtypeStruct((NW, L), jnp.float32),
    ),
    mesh=plsc.VectorSubcoreMesh(
        core_axis_name="c", subcore_axis_name="s",
        num_cores=NC, num_subcores=NS,
    ),
    compiler_params=pltpu.CompilerParams(
        needs_layout_passes=False, use_tc_tiling_on_sc=False),
    scratch_types=(
        pltpu.VMEM((TOK_PER_W,), jnp.int32),
        pltpu.VMEM((TOK_PER_W,), jnp.int32),
        pltpu.VMEM((LSE_PAD,), jnp.float32),
        pltpu.VMEM((CHUNK, VOCAB), jnp.float32),
        pltpu.VMEM((L,), jnp.float32),
        pltpu.SemaphoreType.DMA,
    ),
)
def _sc_gather(table_hbm, idx_hbm, tgt_hbm, lse_hbm,
               out_hbm, psum_hbm,
               idx_v, tgt_v, lse_v, rows_v, acc_v, sem):
    wid = lax.axis_index("s") * NC + lax.axis_index("c")
    base = wid * TOK_PER_W
    pltpu.sync_copy(idx_hbm.at[pl.ds(base, TOK_PER_W)], idx_v)
    pltpu.sync_copy(tgt_hbm.at[pl.ds(base, TOK_PER_W)], tgt_v)
    pltpu.sync_copy(lse_hbm, lse_v)

    def chunk_body(c, acc):
        off = c * CHUNK
        pltpu.async_copy(
            table_hbm.at[idx_v.at[pl.ds(off, CHUNK)]], rows_v, sem).wait()
        pltpu.sync_copy(rows_v, out_hbm.at[pl.ds(base + off, CHUNK)])
        for j in range(CHUNK // L):
            tok = idx_v[pl.ds(off + j * L, L)]
            col = tgt_v[pl.ds(off + j * L, L)]
            lse_tok = plsc.load_gather(lse_v, [tok])
            row_ids = lax.iota(jnp.int32, L) + (j * L)
            tval = plsc.load_gather(rows_v, [row_ids, col])
            acc = acc + (lse_tok - tval)
        return acc

    acc = lax.fori_loop(0, NCHUNK, chunk_body, jnp.zeros((L,), jnp.float32))
    acc_v[...] = jnp.full((L,), jnp.sum(acc), jnp.float32)
    pltpu.sync_copy(acc_v, psum_hbm.at[wid])


def kernel(inputs, targets, table):
    idx_flat = inputs.reshape(N)
    tgt_flat = targets.reshape(N)
    lse_col = _lse_call(table)
    lse_flat = jnp.pad(lse_col[:, 0], (0, LSE_PAD - VOCAB))
    logits_flat, psum = _sc_gather(table, idx_flat, tgt_flat, lse_flat)
    loss = jnp.sum(psum[:, 0]) / N
    return (logits_flat.reshape(B, T, VOCAB), loss)


# speedup vs baseline: 1.4312x; 1.0135x over previous
import functools

import jax
import jax.numpy as jnp
from jax import lax
from jax.experimental import pallas as pl
from jax.experimental.pallas import tpu as pltpu
from jax.experimental.pallas import tpu_sc as plsc

VOCAB = 1000
B = 1024
T = 50
N = B * T
NC = 2
NS = 16
L = 16
NW = NC * NS
TOK_PER_W = N // NW
CHUNK = 32
NCHUNK = TOK_PER_W // CHUNK
NPAIR = NCHUNK // 2
LSE_PAD = 1024


def _lse_body(table_ref, out_ref):
    x = table_ref[...]
    m = jnp.max(x, axis=1, keepdims=True)
    s = jnp.sum(jnp.exp(x - m), axis=1, keepdims=True)
    out_ref[...] = jnp.log(s) + m


_lse_call = pl.pallas_call(
    _lse_body,
    out_shape=jax.ShapeDtypeStruct((VOCAB, 1), jnp.float32),
)


@functools.partial(
    pl.kernel,
    out_type=(
        jax.ShapeDtypeStruct((N, VOCAB), jnp.float32),
        jax.ShapeDtypeStruct((NW, L), jnp.float32),
    ),
    mesh=plsc.VectorSubcoreMesh(
        core_axis_name="c", subcore_axis_name="s",
        num_cores=NC, num_subcores=NS,
    ),
    compiler_params=pltpu.CompilerParams(
        needs_layout_passes=False, use_tc_tiling_on_sc=False),
    scratch_types=(
        pltpu.VMEM((TOK_PER_W,), jnp.int32),
        pltpu.VMEM((TOK_PER_W,), jnp.int32),
        pltpu.VMEM((LSE_PAD,), jnp.float32),
        pltpu.VMEM((CHUNK, VOCAB), jnp.float32),
        pltpu.VMEM((CHUNK, VOCAB), jnp.float32),
        pltpu.VMEM((L,), jnp.float32),
        pltpu.SemaphoreType.DMA,
        pltpu.SemaphoreType.DMA,
        pltpu.SemaphoreType.DMA,
        pltpu.SemaphoreType.DMA,
    ),
)
def _sc_gather(table_hbm, idx_hbm, tgt_hbm, lse_hbm,
               out_hbm, psum_hbm,
               idx_v, tgt_v, lse_v, rows0, rows1, acc_v,
               gsem0, gsem1, osem0, osem1):
    wid = lax.axis_index("s") * NC + lax.axis_index("c")
    base = wid * TOK_PER_W
    pltpu.sync_copy(idx_hbm.at[pl.ds(base, TOK_PER_W)], idx_v)
    pltpu.sync_copy(tgt_hbm.at[pl.ds(base, TOK_PER_W)], tgt_v)
    pltpu.sync_copy(lse_hbm, lse_v)

    def g_start(c, rows, gsem):
        pltpu.async_copy(
            table_hbm.at[idx_v.at[pl.ds(c * CHUNK, CHUNK)]], rows, gsem)

    def g_wait(rows, gsem):
        pltpu.make_async_copy(
            table_hbm.at[idx_v.at[pl.ds(0, CHUNK)]], rows, gsem).wait()

    def o_start(c, rows, osem):
        pltpu.async_copy(rows, out_hbm.at[pl.ds(base + c * CHUNK, CHUNK)], osem)

    def o_wait(rows, osem):
        pltpu.make_async_copy(
            rows, out_hbm.at[pl.ds(base, CHUNK)], osem).wait()

    def compute(c, rows, acc):
        off = c * CHUNK
        for j in range(CHUNK // L):
            tok = idx_v[pl.ds(off + j * L, L)]
            col = tgt_v[pl.ds(off + j * L, L)]
            lse_tok = plsc.load_gather(lse_v, [tok])
            row_ids = lax.iota(jnp.int32, L) + (j * L)
            tval = plsc.load_gather(rows, [row_ids, col])
            acc = acc + (lse_tok - tval)
        return acc

    g_start(0, rows0, gsem0)
    g_start(1, rows1, gsem1)

    def pair_body(g, acc):
        c0 = 2 * g
        for c, rows, gsem, osem in (
                (c0, rows0, gsem0, osem0), (c0 + 1, rows1, gsem1, osem1)):
            g_wait(rows, gsem)
            acc = compute(c, rows, acc)
            o_start(c, rows, osem)

            @pl.when(g < NPAIR - 1)
            def _():
                o_wait(rows, osem)
                g_start(c + 2, rows, gsem)
        return acc

    acc = lax.fori_loop(0, NPAIR, pair_body, jnp.zeros((L,), jnp.float32))
    o_wait(rows0, osem0)
    o_wait(rows1, osem1)
    acc_v[...] = jnp.full((L,), jnp.sum(acc), jnp.float32)
    pltpu.sync_copy(acc_v, psum_hbm.at[wid])


def kernel(inputs, targets, table):
    idx_flat = inputs.reshape(N)
    tgt_flat = targets.reshape(N)
    lse_col = _lse_call(table)
    lse_flat = jnp.pad(lse_col[:, 0], (0, LSE_PAD - VOCAB))
    logits_flat, psum = _sc_gather(table, idx_flat, tgt_flat, lse_flat)
    loss = jnp.sum(psum[:, 0]) / N
    return (logits_flat.reshape(B, T, VOCAB), loss)
